# Initial kernel scaffold; baseline (speedup 1.0000x reference)
import functools

import jax
import jax.numpy as jnp
from jax import lax
from jax.experimental import pallas as pl
from jax.experimental.pallas import tpu as pltpu
from jax.experimental.pallas import tpu_sc as plsc

N = 10000
NP = 10240
E = 320000
EP = 327680
D = 128
G = 64
NCLS = 18
NC = 2
NS = 16
NW = NC * NS
CH = 128
CPT = EP // NW // CH
RPT = NP // NS
RCH = RPT // CH
BN = 512
NBLK = NP // BN

_mesh = plsc.VectorSubcoreMesh(core_axis_name="c", subcore_axis_name="s")

_f32 = jnp.float32


@functools.partial(
    pl.kernel,
    out_type=jax.ShapeDtypeStruct((NC, NP, 16), _f32),
    mesh=_mesh,
    scratch_types=[
        pltpu.VMEM_SHARED((NP, 16), _f32),
        pltpu.VMEM((CPT, CH), jnp.int32),
        pltpu.VMEM((CH, 16), _f32),
        pltpu.VMEM((RPT, 16), _f32),
    ],
)
def _sc_deg(dst_hbm, ones_hbm, zeros_hbm, out_hbm, acc, dstv, onesv, tmp):
    c = lax.axis_index("c")
    s = lax.axis_index("s")
    wid = s * NC + c
    base_r = s * RPT
    pltpu.sync_copy(zeros_hbm, tmp)
    pltpu.sync_copy(tmp, acc.at[pl.ds(base_r, RPT)])
    pltpu.sync_copy(ones_hbm, onesv)
    pltpu.sync_copy(dst_hbm.at[pl.ds(wid * CPT, CPT)], dstv)
    plsc.subcore_barrier()

    def body(j, carry):
        pltpu.sync_copy(onesv, acc.at[dstv.at[j]], add=True)
        return carry

    lax.fori_loop(0, CPT, body, 0)
    plsc.subcore_barrier()
    pltpu.sync_copy(acc.at[pl.ds(base_r, RPT)], tmp)
    pltpu.sync_copy(tmp, out_hbm.at[c, pl.ds(base_r, RPT)])


@functools.partial(
    pl.kernel,
    out_type=jax.ShapeDtypeStruct((NC, NP, D), _f32),
    mesh=_mesh,
    scratch_types=[
        pltpu.VMEM_SHARED((NP, D), _f32),
        pltpu.VMEM((CPT, CH), jnp.int32),
        pltpu.VMEM((CPT, CH), jnp.int32),
        pltpu.VMEM((CH, D), _f32),
        pltpu.VMEM((CH, D), _f32),
        pltpu.SemaphoreType.DMA,
    ],
)
def _sc_scatter(ys_hbm, src_hbm, dst_hbm, out_hbm, acc, srcv, dstv, rows, tmp, sem):
    c = lax.axis_index("c")
    s = lax.axis_index("s")
    wid = s * NC + c
    base_r = s * RPT
    for k in range(RCH):
        pltpu.sync_copy(ys_hbm.at[pl.ds(base_r + k * CH, CH)], tmp)
        pltpu.sync_copy(tmp, acc.at[pl.ds(base_r + k * CH, CH)])
    pltpu.sync_copy(src_hbm.at[pl.ds(wid * CPT, CPT)], srcv)
    pltpu.sync_copy(dst_hbm.at[pl.ds(wid * CPT, CPT)], dstv)
    plsc.subcore_barrier()

    def body(j, carry):
        pltpu.async_copy(ys_hbm.at[srcv.at[j]], rows, sem).wait()
        pltpu.sync_copy(rows, acc.at[dstv.at[j]], add=True)
        return carry

    lax.fori_loop(0, CPT, body, 0)
    plsc.subcore_barrier()
    for k in range(RCH):
        pltpu.sync_copy(acc.at[pl.ds(base_r + k * CH, CH)], tmp)
        pltpu.sync_copy(tmp, out_hbm.at[c, pl.ds(base_r + k * CH, CH)])


def _tc_first_body(x_ref, d0_ref, d1_ref, w_ref, ysc_ref, isqb_ref):
    d = d0_ref[:, 0:1] + d1_ref[:, 0:1] + 1.0
    isqb = jnp.broadcast_to(lax.rsqrt(d), (BN, D))
    isqb_ref[...] = isqb
    ysc_ref[...] = jnp.dot(x_ref[...] * isqb, w_ref[...],
                           preferred_element_type=_f32)


def _tc_mid_body(p0_ref, p1_ref, ys_ref, isq_ref, b_ref, w_ref, out_ref):
    isqb = isq_ref[...]
    h = (p0_ref[...] + p1_ref[...] - ys_ref[...]) * isqb + b_ref[0:1, :]
    h = jnp.maximum(h, 0.0)
    out_ref[...] = jnp.dot(h * isqb, w_ref[...], preferred_element_type=_f32)


def _tc_final_body(p0_ref, p1_ref, ys_ref, isq_ref, b_ref, batch_ref,
                   gam_ref, bet_ref, f1w_ref, f1b_ref, f2w_ref, f2b_ref,
                   out_ref, pooled):
    i = pl.program_id(0)
    h = (p0_ref[...] + p1_ref[...] - ys_ref[...]) * isq_ref[...] + b_ref[0:1, :]
    gi = lax.broadcasted_iota(jnp.int32, (BN, G), 1)
    oh = (batch_ref[:, :G] == gi).astype(_f32)
    contrib = lax.dot_general(oh, h, (((0,), (0,)), ((), ())),
                              preferred_element_type=_f32)

    @pl.when(i == 0)
    def _():
        pooled[...] = jnp.zeros_like(pooled)

    pooled[...] += contrib

    @pl.when(i == NBLK - 1)
    def _():
        p = pooled[...]
        mean = jnp.mean(p, axis=0, keepdims=True)
        var = jnp.mean((p - mean) ** 2, axis=0, keepdims=True)
        hb = (p - mean) * lax.rsqrt(var + 1e-5) * gam_ref[0:1, :] + bet_ref[0:1, :]
        h1 = jnp.dot(hb, f1w_ref[...], preferred_element_type=_f32) + f1b_ref[0:1, :]
        h1 = jnp.maximum(h1, 0.0)
        logits = jnp.dot(h1, f2w_ref[...], preferred_element_type=_f32) + f2b_ref[0:1, :]
        col = lax.broadcasted_iota(jnp.int32, (G, D), 1)
        logits = jnp.where(col < NCLS, logits, -1e30)
        m = jnp.max(logits, axis=1, keepdims=True)
        lse = jnp.log(jnp.sum(jnp.exp(logits - m), axis=1, keepdims=True)) + m
        out_ref[...] = logits - lse


def _row_spec(i):
    return (i, 0)


def _fixed_spec(i):
    return (0, 0)


_tc_first = pl.pallas_call(
    _tc_first_body,
    grid=(NBLK,),
    in_specs=[
        pl.BlockSpec((BN, D), _row_spec),
        pl.BlockSpec((BN, 16), _row_spec),
        pl.BlockSpec((BN, 16), _row_spec),
        pl.BlockSpec((D, D), _fixed_spec),
    ],
    out_specs=[pl.BlockSpec((BN, D), _row_spec), pl.BlockSpec((BN, D), _row_spec)],
    out_shape=[jax.ShapeDtypeStruct((NP, D), _f32), jax.ShapeDtypeStruct((NP, D), _f32)],
)

_tc_mid = pl.pallas_call(
    _tc_mid_body,
    grid=(NBLK,),
    in_specs=[
        pl.BlockSpec((BN, D), _row_spec),
        pl.BlockSpec((BN, D), _row_spec),
        pl.BlockSpec((BN, D), _row_spec),
        pl.BlockSpec((BN, D), _row_spec),
        pl.BlockSpec((8, D), _fixed_spec),
        pl.BlockSpec((D, D), _fixed_spec),
    ],
    out_specs=pl.BlockSpec((BN, D), _row_spec),
    out_shape=jax.ShapeDtypeStruct((NP, D), _f32),
)

_tc_final = pl.pallas_call(
    _tc_final_body,
    grid=(NBLK,),
    in_specs=[
        pl.BlockSpec((BN, D), _row_spec),
        pl.BlockSpec((BN, D), _row_spec),
        pl.BlockSpec((BN, D), _row_spec),
        pl.BlockSpec((BN, D), _row_spec),
        pl.BlockSpec((8, D), _fixed_spec),
        pl.BlockSpec((BN, D), _row_spec),
        pl.BlockSpec((8, D), _fixed_spec),
        pl.BlockSpec((8, D), _fixed_spec),
        pl.BlockSpec((D, D), _fixed_spec),
        pl.BlockSpec((8, D), _fixed_spec),
        pl.BlockSpec((D, D), _fixed_spec),
        pl.BlockSpec((8, D), _fixed_spec),
    ],
    out_specs=pl.BlockSpec((G, D), _fixed_spec),
    out_shape=jax.ShapeDtypeStruct((G, D), _f32),
    scratch_shapes=[pltpu.VMEM((G, D), _f32)],
)


def _bc8(v):
    return jnp.broadcast_to(v[None, :], (8, v.shape[0]))


@jax.jit
def kernel(x, edge_index, batch, W_first, b_first, W_hidden, b_hidden,
           W_last, b_last, bn_gamma, bn_beta, fc1_W, fc1_b, fc2_W, fc2_b):
    i32 = jnp.int32
    pad_e = jnp.full((EP - E,), N, i32)
    srcp = jnp.concatenate([edge_index[0], pad_e]).reshape(EP // CH, CH)
    dstp = jnp.concatenate([edge_index[1], pad_e]).reshape(EP // CH, CH)
    x_pad = jnp.pad(x, ((0, NP - N), (0, 0)))
    batch_bc = jnp.broadcast_to(
        jnp.pad(batch, (0, NP - N), constant_values=G)[:, None], (NP, D))
    ones16 = jnp.ones((CH, 16), _f32)
    zeros16 = jnp.zeros((RPT, 16), _f32)

    degp = _sc_deg(dstp, ones16, zeros16)
    ysc, isqb = _tc_first(x_pad, degp[0], degp[1], W_first)

    bs = [b_first] + [b_hidden[k] for k in range(4)]
    ws = [W_hidden[k] for k in range(4)] + [W_last]
    for l in range(5):
        p = _sc_scatter(ysc, srcp, dstp)
        ysc = _tc_mid(p[0], p[1], ysc, isqb, _bc8(bs[l]), ws[l])
    p = _sc_scatter(ysc, srcp, dstp)

    f2w_pad = jnp.zeros((D, D), _f32).at[:, :NCLS].set(fc2_W)
    f2b_pad = jnp.zeros((D,), _f32).at[:NCLS].set(fc2_b)
    res = _tc_final(p[0], p[1], ysc, isqb, _bc8(b_last), batch_bc,
                    _bc8(bn_gamma), _bc8(bn_beta), fc1_W, _bc8(fc1_b),
                    f2w_pad, _bc8(f2b_pad))
    return res[:, :NCLS]


# speedup vs baseline: 5.1044x; 5.1044x over previous
import functools

import jax
import jax.numpy as jnp
from jax import lax
from jax.experimental import pallas as pl
from jax.experimental.pallas import tpu as pltpu
from jax.experimental.pallas import tpu_sc as plsc

N = 10000
NP = 10240
E = 320000
EP = 327680
D = 128
G = 64
NCLS = 18
NC = 2
NS = 16
NW = NC * NS
CH = 128
CPT = EP // NW // CH
RPT = NP // NS
RCH = RPT // CH
BN = 512
NBLK = NP // BN

_mesh = plsc.VectorSubcoreMesh(core_axis_name="c", subcore_axis_name="s")

_f32 = jnp.float32


@functools.partial(
    pl.kernel,
    out_type=jax.ShapeDtypeStruct((NC, NP, D), _f32),
    mesh=_mesh,
    scratch_types=[
        pltpu.VMEM_SHARED((NP, D), _f32),
        pltpu.VMEM((CPT, CH), jnp.int32),
        pltpu.VMEM((CH, D), _f32),
    ],
)
def _sc_deg(dst_hbm, ones_hbm, out_hbm, acc, dstv, onesv):
    c = lax.axis_index("c")
    s = lax.axis_index("s")
    wid = s * NC + c
    base_r = s * RPT
    pltpu.sync_copy(ones_hbm, onesv)
    for k in range(RCH):
        pltpu.sync_copy(onesv, acc.at[pl.ds(base_r + k * CH, CH)])
    pltpu.sync_copy(dst_hbm.at[pl.ds(wid * CPT, CPT)], dstv)
    plsc.subcore_barrier()

    def body(j, carry):
        pltpu.sync_copy(onesv, acc.at[dstv.at[j]], add=True)
        return carry

    lax.fori_loop(0, CPT, body, 0)
    plsc.subcore_barrier()
    for k in range(RCH):
        pltpu.sync_copy(acc.at[pl.ds(base_r + k * CH, CH)], onesv)
        pltpu.sync_copy(onesv, out_hbm.at[c, pl.ds(base_r + k * CH, CH)])


@functools.partial(
    pl.kernel,
    out_type=jax.ShapeDtypeStruct((NC, NP, D), _f32),
    mesh=_mesh,
    scratch_types=[
        pltpu.VMEM_SHARED((NP, D), _f32),
        pltpu.VMEM((CPT, CH), jnp.int32),
        pltpu.VMEM((CPT, CH), jnp.int32),
        pltpu.VMEM((CH, D), _f32),
        pltpu.SemaphoreType.DMA,
    ],
)
def _sc_scatter(ys_hbm, src_hbm, dst_hbm, out_hbm, acc, srcv, dstv, rows, sem):
    c = lax.axis_index("c")
    s = lax.axis_index("s")
    wid = s * NC + c
    base_r = s * RPT
    for k in range(RCH):
        pltpu.sync_copy(ys_hbm.at[pl.ds(base_r + k * CH, CH)], rows)
        pltpu.sync_copy(rows, acc.at[pl.ds(base_r + k * CH, CH)])
    pltpu.sync_copy(src_hbm.at[pl.ds(wid * CPT, CPT)], srcv)
    pltpu.sync_copy(dst_hbm.at[pl.ds(wid * CPT, CPT)], dstv)
    plsc.subcore_barrier()

    def body(j, carry):
        pltpu.async_copy(ys_hbm.at[srcv.at[j]], rows, sem).wait()
        pltpu.sync_copy(rows, acc.at[dstv.at[j]], add=True)
        return carry

    lax.fori_loop(0, CPT, body, 0)
    plsc.subcore_barrier()
    for k in range(RCH):
        pltpu.sync_copy(acc.at[pl.ds(base_r + k * CH, CH)], rows)
        pltpu.sync_copy(rows, out_hbm.at[c, pl.ds(base_r + k * CH, CH)])


def _tc_first_body(x_ref, d0_ref, d1_ref, w_ref, ysc_ref, isqb_ref):
    d = d0_ref[:, 0:1] + d1_ref[:, 0:1] - 1.0
    isqb = jnp.broadcast_to(lax.rsqrt(d), (BN, D))
    isqb_ref[...] = isqb
    ysc_ref[...] = jnp.dot(x_ref[...] * isqb, w_ref[...],
                           preferred_element_type=_f32)


def _tc_mid_body(p0_ref, p1_ref, ys_ref, isq_ref, b_ref, w_ref, out_ref):
    isqb = isq_ref[...]
    h = (p0_ref[...] + p1_ref[...] - ys_ref[...]) * isqb + b_ref[0:1, :]
    h = jnp.maximum(h, 0.0)
    out_ref[...] = jnp.dot(h * isqb, w_ref[...], preferred_element_type=_f32)


def _tc_final_body(p0_ref, p1_ref, ys_ref, isq_ref, b_ref, batch_ref,
                   gam_ref, bet_ref, f1w_ref, f1b_ref, f2w_ref, f2b_ref,
                   out_ref, pooled):
    i = pl.program_id(0)
    h = (p0_ref[...] + p1_ref[...] - ys_ref[...]) * isq_ref[...] + b_ref[0:1, :]
    gi = lax.broadcasted_iota(jnp.int32, (BN, G), 1)
    oh = (batch_ref[:, :G] == gi).astype(_f32)
    contrib = lax.dot_general(oh, h, (((0,), (0,)), ((), ())),
                              preferred_element_type=_f32)

    @pl.when(i == 0)
    def _():
        pooled[...] = jnp.zeros_like(pooled)

    pooled[...] += contrib

    @pl.when(i == NBLK - 1)
    def _():
        p = pooled[...]
        mean = jnp.mean(p, axis=0, keepdims=True)
        var = jnp.mean((p - mean) ** 2, axis=0, keepdims=True)
        hb = (p - mean) * lax.rsqrt(var + 1e-5) * gam_ref[0:1, :] + bet_ref[0:1, :]
        h1 = jnp.dot(hb, f1w_ref[...], preferred_element_type=_f32) + f1b_ref[0:1, :]
        h1 = jnp.maximum(h1, 0.0)
        logits = jnp.dot(h1, f2w_ref[...], preferred_element_type=_f32) + f2b_ref[0:1, :]
        col = lax.broadcasted_iota(jnp.int32, (G, D), 1)
        logits = jnp.where(col < NCLS, logits, -1e30)
        m = jnp.max(logits, axis=1, keepdims=True)
        lse = jnp.log(jnp.sum(jnp.exp(logits - m), axis=1, keepdims=True)) + m
        out_ref[...] = logits - lse


def _row_spec(i):
    return (i, 0)


def _fixed_spec(i):
    return (0, 0)


_tc_first = pl.pallas_call(
    _tc_first_body,
    grid=(NBLK,),
    in_specs=[
        pl.BlockSpec((BN, D), _row_spec),
        pl.BlockSpec((BN, D), _row_spec),
        pl.BlockSpec((BN, D), _row_spec),
        pl.BlockSpec((D, D), _fixed_spec),
    ],
    out_specs=[pl.BlockSpec((BN, D), _row_spec), pl.BlockSpec((BN, D), _row_spec)],
    out_shape=[jax.ShapeDtypeStruct((NP, D), _f32), jax.ShapeDtypeStruct((NP, D), _f32)],
)

_tc_mid = pl.pallas_call(
    _tc_mid_body,
    grid=(NBLK,),
    in_specs=[
        pl.BlockSpec((BN, D), _row_spec),
        pl.BlockSpec((BN, D), _row_spec),
        pl.BlockSpec((BN, D), _row_spec),
        pl.BlockSpec((BN, D), _row_spec),
        pl.BlockSpec((8, D), _fixed_spec),
        pl.BlockSpec((D, D), _fixed_spec),
    ],
    out_specs=pl.BlockSpec((BN, D), _row_spec),
    out_shape=jax.ShapeDtypeStruct((NP, D), _f32),
)

_tc_final = pl.pallas_call(
    _tc_final_body,
    grid=(NBLK,),
    in_specs=[
        pl.BlockSpec((BN, D), _row_spec),
        pl.BlockSpec((BN, D), _row_spec),
        pl.BlockSpec((BN, D), _row_spec),
        pl.BlockSpec((BN, D), _row_spec),
        pl.BlockSpec((8, D), _fixed_spec),
        pl.BlockSpec((BN, D), _row_spec),
        pl.BlockSpec((8, D), _fixed_spec),
        pl.BlockSpec((8, D), _fixed_spec),
        pl.BlockSpec((D, D), _fixed_spec),
        pl.BlockSpec((8, D), _fixed_spec),
        pl.BlockSpec((D, D), _fixed_spec),
        pl.BlockSpec((8, D), _fixed_spec),
    ],
    out_specs=pl.BlockSpec((G, D), _fixed_spec),
    out_shape=jax.ShapeDtypeStruct((G, D), _f32),
    scratch_shapes=[pltpu.VMEM((G, D), _f32)],
)


def _bc8(v):
    return jnp.broadcast_to(v[None, :], (8, v.shape[0]))


@jax.jit
def kernel(x, edge_index, batch, W_first, b_first, W_hidden, b_hidden,
           W_last, b_last, bn_gamma, bn_beta, fc1_W, fc1_b, fc2_W, fc2_b):
    i32 = jnp.int32
    pad_e = jnp.full((EP - E,), N, i32)
    srcp = jnp.concatenate([edge_index[0], pad_e]).reshape(EP // CH, CH)
    dstp = jnp.concatenate([edge_index[1], pad_e]).reshape(EP // CH, CH)
    x_pad = jnp.pad(x, ((0, NP - N), (0, 0)))
    batch_bc = jnp.broadcast_to(
        jnp.pad(batch, (0, NP - N), constant_values=G)[:, None], (NP, D))
    ones128 = jnp.ones((CH, D), _f32)

    degp = _sc_deg(dstp, ones128)
    ysc, isqb = _tc_first(x_pad, degp[0], degp[1], W_first)

    bs = [b_first] + [b_hidden[k] for k in range(4)]
    ws = [W_hidden[k] for k in range(4)] + [W_last]
    for l in range(5):
        p = _sc_scatter(ysc, srcp, dstp)
        ysc = _tc_mid(p[0], p[1], ysc, isqb, _bc8(bs[l]), ws[l])
    p = _sc_scatter(ysc, srcp, dstp)

    f2w_pad = jnp.zeros((D, D), _f32).at[:, :NCLS].set(fc2_W)
    f2b_pad = jnp.zeros((D,), _f32).at[:NCLS].set(fc2_b)
    res = _tc_final(p[0], p[1], ysc, isqb, _bc8(b_last), batch_bc,
                    _bc8(bn_gamma), _bc8(bn_beta), fc1_W, _bc8(fc1_b),
                    f2w_pad, _bc8(f2b_pad))
    return res[:, :NCLS]


# speedup vs baseline: 5.6542x; 1.1077x over previous
import functools

import jax
import jax.numpy as jnp
from jax import lax
from jax.experimental import pallas as pl
from jax.experimental.pallas import tpu as pltpu
from jax.experimental.pallas import tpu_sc as plsc

N = 10000
NP = 10240
E = 320000
EP = 327680
D = 128
G = 64
NCLS = 18
NC = 2
NS = 16
NW = NC * NS
CH = 128
CPT = EP // NW // CH
RPT = NP // NS
RCH = RPT // CH
BN = 512
NBLK = NP // BN

_mesh = plsc.VectorSubcoreMesh(core_axis_name="c", subcore_axis_name="s")

_f32 = jnp.float32


@functools.partial(
    pl.kernel,
    out_type=jax.ShapeDtypeStruct((NC, NP, D), _f32),
    mesh=_mesh,
    scratch_types=[
        pltpu.VMEM_SHARED((NP, D), _f32),
        pltpu.VMEM((CPT, CH), jnp.int32),
        pltpu.VMEM((CH, D), _f32),
    ],
)
def _sc_deg(dst_hbm, ones_hbm, out_hbm, acc, dstv, onesv):
    c = lax.axis_index("c")
    s = lax.axis_index("s")
    wid = s * NC + c
    base_r = s * RPT
    pltpu.sync_copy(ones_hbm, onesv)
    for k in range(RCH):
        pltpu.sync_copy(onesv, acc.at[pl.ds(base_r + k * CH, CH)])
    pltpu.sync_copy(dst_hbm.at[pl.ds(wid * CPT, CPT)], dstv)
    plsc.subcore_barrier()

    def body(j, carry):
        pltpu.sync_copy(onesv, acc.at[dstv.at[j]], add=True)
        return carry

    lax.fori_loop(0, CPT, body, 0)
    plsc.subcore_barrier()
    for k in range(RCH):
        pltpu.sync_copy(acc.at[pl.ds(base_r + k * CH, CH)], onesv)
        pltpu.sync_copy(onesv, out_hbm.at[c, pl.ds(base_r + k * CH, CH)])


HC = CPT // 2


@functools.partial(
    pl.kernel,
    out_type=jax.ShapeDtypeStruct((NC, NP, D), _f32),
    mesh=_mesh,
    scratch_types=[
        pltpu.VMEM_SHARED((NP, D), _f32),
        pltpu.VMEM((HC, CH), jnp.int32),
        pltpu.VMEM((HC, CH), jnp.int32),
        pltpu.VMEM((CH, D), _f32),
        pltpu.VMEM((CH, D), _f32),
        pltpu.SemaphoreType.DMA,
        pltpu.SemaphoreType.DMA,
        pltpu.SemaphoreType.DMA,
        pltpu.SemaphoreType.DMA,
    ],
)
def _sc_scatter(ys_hbm, src_hbm, dst_hbm, out_hbm, acc,
                srcv, dstv, rows0, rows1, g0, g1, s0, s1):
    c = lax.axis_index("c")
    s = lax.axis_index("s")
    wid = s * NC + c
    base_r = s * RPT
    rows = (rows0, rows1)
    gsem = (g0, g1)
    ssem = (s0, s1)
    for k in range(RCH):
        pltpu.sync_copy(ys_hbm.at[pl.ds(base_r + k * CH, CH)], rows0)
        pltpu.sync_copy(rows0, acc.at[pl.ds(base_r + k * CH, CH)])
    plsc.subcore_barrier()

    for h in range(2):
        cbase = wid * CPT + h * HC
        pltpu.sync_copy(src_hbm.at[pl.ds(cbase, HC)], srcv)
        pltpu.sync_copy(dst_hbm.at[pl.ds(cbase, HC)], dstv)
        for b in range(2):
            pltpu.async_copy(ys_hbm.at[srcv.at[b]], rows[b], gsem[b])

        def outer(jo, carry):
            for b in range(2):
                ch = 2 * jo + b
                pltpu.make_async_copy(ys_hbm.at[srcv.at[ch]],
                                      rows[b], gsem[b]).wait()
                sc = pltpu.async_copy(rows[b], acc.at[dstv.at[ch]],
                                      ssem[b], add=True)
                nxt = jnp.minimum(ch + 2, HC - 1)
                sc.wait()

                @pl.when(ch + 2 < HC)
                def _():
                    pltpu.async_copy(ys_hbm.at[srcv.at[nxt]], rows[b], gsem[b])
            return carry

        lax.fori_loop(0, HC // 2, outer, 0)
    plsc.subcore_barrier()
    for k in range(RCH):
        pltpu.sync_copy(acc.at[pl.ds(base_r + k * CH, CH)], rows0)
        pltpu.sync_copy(rows0, out_hbm.at[c, pl.ds(base_r + k * CH, CH)])


def _tc_first_body(x_ref, d0_ref, d1_ref, w_ref, ysc_ref, isqb_ref):
    d = d0_ref[:, 0:1] + d1_ref[:, 0:1] - 1.0
    isqb = jnp.broadcast_to(lax.rsqrt(d), (BN, D))
    isqb_ref[...] = isqb
    ysc_ref[...] = jnp.dot(x_ref[...] * isqb, w_ref[...],
                           preferred_element_type=_f32)


def _tc_mid_body(p0_ref, p1_ref, ys_ref, isq_ref, b_ref, w_ref, out_ref):
    isqb = isq_ref[...]
    h = (p0_ref[...] + p1_ref[...] - ys_ref[...]) * isqb + b_ref[0:1, :]
    h = jnp.maximum(h, 0.0)
    out_ref[...] = jnp.dot(h * isqb, w_ref[...], preferred_element_type=_f32)


def _tc_final_body(p0_ref, p1_ref, ys_ref, isq_ref, b_ref, batch_ref,
                   gam_ref, bet_ref, f1w_ref, f1b_ref, f2w_ref, f2b_ref,
                   out_ref, pooled):
    i = pl.program_id(0)
    h = (p0_ref[...] + p1_ref[...] - ys_ref[...]) * isq_ref[...] + b_ref[0:1, :]
    gi = lax.broadcasted_iota(jnp.int32, (BN, G), 1)
    oh = (batch_ref[:, :G] == gi).astype(_f32)
    contrib = lax.dot_general(oh, h, (((0,), (0,)), ((), ())),
                              preferred_element_type=_f32)

    @pl.when(i == 0)
    def _():
        pooled[...] = jnp.zeros_like(pooled)

    pooled[...] += contrib

    @pl.when(i == NBLK - 1)
    def _():
        p = pooled[...]
        mean = jnp.mean(p, axis=0, keepdims=True)
        var = jnp.mean((p - mean) ** 2, axis=0, keepdims=True)
        hb = (p - mean) * lax.rsqrt(var + 1e-5) * gam_ref[0:1, :] + bet_ref[0:1, :]
        h1 = jnp.dot(hb, f1w_ref[...], preferred_element_type=_f32) + f1b_ref[0:1, :]
        h1 = jnp.maximum(h1, 0.0)
        logits = jnp.dot(h1, f2w_ref[...], preferred_element_type=_f32) + f2b_ref[0:1, :]
        col = lax.broadcasted_iota(jnp.int32, (G, D), 1)
        logits = jnp.where(col < NCLS, logits, -1e30)
        m = jnp.max(logits, axis=1, keepdims=True)
        lse = jnp.log(jnp.sum(jnp.exp(logits - m), axis=1, keepdims=True)) + m
        out_ref[...] = logits - lse


def _row_spec(i):
    return (i, 0)


def _fixed_spec(i):
    return (0, 0)


_tc_first = pl.pallas_call(
    _tc_first_body,
    grid=(NBLK,),
    in_specs=[
        pl.BlockSpec((BN, D), _row_spec),
        pl.BlockSpec((BN, D), _row_spec),
        pl.BlockSpec((BN, D), _row_spec),
        pl.BlockSpec((D, D), _fixed_spec),
    ],
    out_specs=[pl.BlockSpec((BN, D), _row_spec), pl.BlockSpec((BN, D), _row_spec)],
    out_shape=[jax.ShapeDtypeStruct((NP, D), _f32), jax.ShapeDtypeStruct((NP, D), _f32)],
)

_tc_mid = pl.pallas_call(
    _tc_mid_body,
    grid=(NBLK,),
    in_specs=[
        pl.BlockSpec((BN, D), _row_spec),
        pl.BlockSpec((BN, D), _row_spec),
        pl.BlockSpec((BN, D), _row_spec),
        pl.BlockSpec((BN, D), _row_spec),
        pl.BlockSpec((8, D), _fixed_spec),
        pl.BlockSpec((D, D), _fixed_spec),
    ],
    out_specs=pl.BlockSpec((BN, D), _row_spec),
    out_shape=jax.ShapeDtypeStruct((NP, D), _f32),
)

_tc_final = pl.pallas_call(
    _tc_final_body,
    grid=(NBLK,),
    in_specs=[
        pl.BlockSpec((BN, D), _row_spec),
        pl.BlockSpec((BN, D), _row_spec),
        pl.BlockSpec((BN, D), _row_spec),
        pl.BlockSpec((BN, D), _row_spec),
        pl.BlockSpec((8, D), _fixed_spec),
        pl.BlockSpec((BN, D), _row_spec),
        pl.BlockSpec((8, D), _fixed_spec),
        pl.BlockSpec((8, D), _fixed_spec),
        pl.BlockSpec((D, D), _fixed_spec),
        pl.BlockSpec((8, D), _fixed_spec),
        pl.BlockSpec((D, D), _fixed_spec),
        pl.BlockSpec((8, D), _fixed_spec),
    ],
    out_specs=pl.BlockSpec((G, D), _fixed_spec),
    out_shape=jax.ShapeDtypeStruct((G, D), _f32),
    scratch_shapes=[pltpu.VMEM((G, D), _f32)],
)


def _bc8(v):
    return jnp.broadcast_to(v[None, :], (8, v.shape[0]))


@jax.jit
def kernel(x, edge_index, batch, W_first, b_first, W_hidden, b_hidden,
           W_last, b_last, bn_gamma, bn_beta, fc1_W, fc1_b, fc2_W, fc2_b):
    i32 = jnp.int32
    pad_e = jnp.full((EP - E,), N, i32)
    srcp = jnp.concatenate([edge_index[0], pad_e]).reshape(EP // CH, CH)
    dstp = jnp.concatenate([edge_index[1], pad_e]).reshape(EP // CH, CH)
    x_pad = jnp.pad(x, ((0, NP - N), (0, 0)))
    batch_bc = jnp.broadcast_to(
        jnp.pad(batch, (0, NP - N), constant_values=G)[:, None], (NP, D))
    ones128 = jnp.ones((CH, D), _f32)

    degp = _sc_deg(dstp, ones128)
    ysc, isqb = _tc_first(x_pad, degp[0], degp[1], W_first)

    bs = [b_first] + [b_hidden[k] for k in range(4)]
    ws = [W_hidden[k] for k in range(4)] + [W_last]
    for l in range(5):
        p = _sc_scatter(ysc, srcp, dstp)
        ysc = _tc_mid(p[0], p[1], ysc, isqb, _bc8(bs[l]), ws[l])
    p = _sc_scatter(ysc, srcp, dstp)

    f2w_pad = jnp.zeros((D, D), _f32).at[:, :NCLS].set(fc2_W)
    f2b_pad = jnp.zeros((D,), _f32).at[:NCLS].set(fc2_b)
    res = _tc_final(p[0], p[1], ysc, isqb, _bc8(b_last), batch_bc,
                    _bc8(bn_gamma), _bc8(bn_beta), fc1_W, _bc8(fc1_b),
                    f2w_pad, _bc8(f2b_pad))
    return res[:, :NCLS]


# speedup vs baseline: 6.2777x; 1.1103x over previous
import functools

import jax
import jax.numpy as jnp
from jax import lax
from jax.experimental import pallas as pl
from jax.experimental.pallas import tpu as pltpu
from jax.experimental.pallas import tpu_sc as plsc

N = 10000
NP = 10240
E = 320000
EP = 327680
D = 128
G = 64
NCLS = 18
NC = 2
NS = 16
NW = NC * NS
CH = 128
CPT = EP // NW // CH
RPT = NP // NS
RCH = RPT // CH
BN = 512
NBLK = NP // BN

_mesh = plsc.VectorSubcoreMesh(core_axis_name="c", subcore_axis_name="s")

_f32 = jnp.float32


@functools.partial(
    pl.kernel,
    out_type=jax.ShapeDtypeStruct((NW, NP // 16, 16), _f32),
    mesh=_mesh,
    scratch_types=[
        pltpu.VMEM((NP // 16, 16), _f32),
        pltpu.VMEM((CPT, CH), jnp.int32),
    ],
)
def _sc_deg(dst_hbm, zeros_hbm, out_hbm, hist, dstv):
    c = lax.axis_index("c")
    s = lax.axis_index("s")
    wid = s * NC + c
    pltpu.sync_copy(zeros_hbm, hist)
    pltpu.sync_copy(dst_hbm.at[pl.ds(wid * CPT, CPT)], dstv)
    iota16 = lax.iota(jnp.int32, 16)

    def body(j, carry):
        for g in range(CH // 16):
            v = dstv[j, pl.ds(g * 16, 16)]
            for k in range(16):
                idx = v[k]
                oh = jnp.where(iota16 == (idx & 15), 1.0, 0.0).astype(_f32)
                plsc.addupdate(hist.at[idx >> 4], oh)
        return carry

    lax.fori_loop(0, CPT, body, 0)
    pltpu.sync_copy(hist, out_hbm.at[wid])


CH3 = 64
CPT3 = EP // NW // CH3
QC3 = CPT3 // 4
NB = 4


@functools.partial(
    pl.kernel,
    out_type=jax.ShapeDtypeStruct((NC, NP, D), _f32),
    mesh=_mesh,
    scratch_types=[
        pltpu.VMEM_SHARED((NP, D), _f32),
        pltpu.VMEM((QC3, CH3), jnp.int32),
        pltpu.VMEM((QC3, CH3), jnp.int32),
    ] + [pltpu.VMEM((CH3, D), _f32)] * NB
      + [pltpu.SemaphoreType.DMA] * (2 * NB),
)
def _sc_scatter(ys_hbm, src_hbm, dst_hbm, out_hbm, acc, srcv, dstv, *bufs):
    rows = bufs[:NB]
    gsem = bufs[NB:2 * NB]
    ssem = bufs[2 * NB:]
    c = lax.axis_index("c")
    s = lax.axis_index("s")
    wid = s * NC + c
    base_r = s * RPT
    for k in range(RPT // CH3):
        pltpu.sync_copy(ys_hbm.at[pl.ds(base_r + k * CH3, CH3)], rows[0])
        pltpu.sync_copy(rows[0], acc.at[pl.ds(base_r + k * CH3, CH3)])
    plsc.subcore_barrier()

    for h in range(4):
        cbase = wid * CPT3 + h * QC3
        pltpu.sync_copy(src_hbm.at[pl.ds(cbase, QC3)], srcv)
        pltpu.sync_copy(dst_hbm.at[pl.ds(cbase, QC3)], dstv)
        for b in range(2):
            pltpu.async_copy(ys_hbm.at[srcv.at[b]], rows[b], gsem[b])

        def outer(jo, carry):
            for b in range(NB):
                ch = NB * jo + b
                bb = (b + 2) % NB
                nxt = jnp.minimum(ch + 2, QC3 - 1)

                @pl.when(jnp.logical_and(ch >= 2, ch + 2 < QC3))
                def _():
                    pltpu.make_async_copy(rows[bb], acc.at[pl.ds(0, CH3)],
                                          ssem[bb]).wait()
                    pltpu.async_copy(ys_hbm.at[srcv.at[nxt]], rows[bb], gsem[bb])

                @pl.when(jnp.logical_and(ch < 2, ch + 2 < QC3))
                def _():
                    pltpu.async_copy(ys_hbm.at[srcv.at[nxt]], rows[bb], gsem[bb])

                pltpu.make_async_copy(ys_hbm.at[srcv.at[ch]],
                                      rows[b], gsem[b]).wait()
                pltpu.async_copy(rows[b], acc.at[dstv.at[ch]], ssem[b], add=True)
            return carry

        lax.fori_loop(0, QC3 // NB, outer, 0)
        for b in range(NB):
            pltpu.make_async_copy(rows[b], acc.at[pl.ds(0, CH3)],
                                  ssem[b]).wait()
    plsc.subcore_barrier()
    for k in range(RPT // CH3):
        pltpu.sync_copy(acc.at[pl.ds(base_r + k * CH3, CH3)], rows[0])
        pltpu.sync_copy(rows[0], out_hbm.at[c, pl.ds(base_r + k * CH3, CH3)])


def _tc_first_body(x_ref, d_ref, w_ref, ysc_ref, isqb_ref):
    d = jnp.sum(d_ref[:, 0], axis=0)
    iota_r = lax.broadcasted_iota(jnp.int32, (D, D), 0)
    iota_c = lax.broadcasted_iota(jnp.int32, (D, D), 1)
    ident = (iota_r == iota_c).astype(_f32)
    cols = [lax.dot_general(ident, d[a:a + 1, :], (((1,), (1,)), ((), ())),
                            preferred_element_type=_f32)
            for a in range(BN // D)]
    dcol = jnp.concatenate(cols, axis=0)
    isqb = jnp.broadcast_to(lax.rsqrt(dcol + 1.0), (BN, D))
    isqb_ref[...] = isqb
    ysc_ref[...] = jnp.dot(x_ref[...] * isqb, w_ref[...],
                           preferred_element_type=_f32)


def _tc_mid_body(p0_ref, p1_ref, ys_ref, isq_ref, b_ref, w_ref, out_ref):
    isqb = isq_ref[...]
    h = (p0_ref[...] + p1_ref[...] - ys_ref[...]) * isqb + b_ref[0:1, :]
    h = jnp.maximum(h, 0.0)
    out_ref[...] = jnp.dot(h * isqb, w_ref[...], preferred_element_type=_f32)


def _tc_final_body(p0_ref, p1_ref, ys_ref, isq_ref, b_ref, batch_ref,
                   gam_ref, bet_ref, f1w_ref, f1b_ref, f2w_ref, f2b_ref,
                   out_ref, pooled):
    i = pl.program_id(0)
    h = (p0_ref[...] + p1_ref[...] - ys_ref[...]) * isq_ref[...] + b_ref[0:1, :]
    gi = lax.broadcasted_iota(jnp.int32, (BN, G), 1)
    oh = (batch_ref[:, :G] == gi).astype(_f32)
    contrib = lax.dot_general(oh, h, (((0,), (0,)), ((), ())),
                              preferred_element_type=_f32)

    @pl.when(i == 0)
    def _():
        pooled[...] = jnp.zeros_like(pooled)

    pooled[...] += contrib

    @pl.when(i == NBLK - 1)
    def _():
        p = pooled[...]
        mean = jnp.mean(p, axis=0, keepdims=True)
        var = jnp.mean((p - mean) ** 2, axis=0, keepdims=True)
        hb = (p - mean) * lax.rsqrt(var + 1e-5) * gam_ref[0:1, :] + bet_ref[0:1, :]
        h1 = jnp.dot(hb, f1w_ref[...], preferred_element_type=_f32) + f1b_ref[0:1, :]
        h1 = jnp.maximum(h1, 0.0)
        logits = jnp.dot(h1, f2w_ref[...], preferred_element_type=_f32) + f2b_ref[0:1, :]
        col = lax.broadcasted_iota(jnp.int32, (G, D), 1)
        logits = jnp.where(col < NCLS, logits, -1e30)
        m = jnp.max(logits, axis=1, keepdims=True)
        lse = jnp.log(jnp.sum(jnp.exp(logits - m), axis=1, keepdims=True)) + m
        out_ref[...] = logits - lse


def _row_spec(i):
    return (i, 0)


def _fixed_spec(i):
    return (0, 0)


_tc_first = pl.pallas_call(
    _tc_first_body,
    grid=(NBLK,),
    in_specs=[
        pl.BlockSpec((BN, D), _row_spec),
        pl.BlockSpec((NW, 1, BN // D, D), lambda i: (0, i, 0, 0)),
        pl.BlockSpec((D, D), _fixed_spec),
    ],
    out_specs=[pl.BlockSpec((BN, D), _row_spec), pl.BlockSpec((BN, D), _row_spec)],
    out_shape=[jax.ShapeDtypeStruct((NP, D), _f32), jax.ShapeDtypeStruct((NP, D), _f32)],
)

_tc_mid = pl.pallas_call(
    _tc_mid_body,
    grid=(NBLK,),
    in_specs=[
        pl.BlockSpec((BN, D), _row_spec),
        pl.BlockSpec((BN, D), _row_spec),
        pl.BlockSpec((BN, D), _row_spec),
        pl.BlockSpec((BN, D), _row_spec),
        pl.BlockSpec((8, D), _fixed_spec),
        pl.BlockSpec((D, D), _fixed_spec),
    ],
    out_specs=pl.BlockSpec((BN, D), _row_spec),
    out_shape=jax.ShapeDtypeStruct((NP, D), _f32),
)

_tc_final = pl.pallas_call(
    _tc_final_body,
    grid=(NBLK,),
    in_specs=[
        pl.BlockSpec((BN, D), _row_spec),
        pl.BlockSpec((BN, D), _row_spec),
        pl.BlockSpec((BN, D), _row_spec),
        pl.BlockSpec((BN, D), _row_spec),
        pl.BlockSpec((8, D), _fixed_spec),
        pl.BlockSpec((BN, D), _row_spec),
        pl.BlockSpec((8, D), _fixed_spec),
        pl.BlockSpec((8, D), _fixed_spec),
        pl.BlockSpec((D, D), _fixed_spec),
        pl.BlockSpec((8, D), _fixed_spec),
        pl.BlockSpec((D, D), _fixed_spec),
        pl.BlockSpec((8, D), _fixed_spec),
    ],
    out_specs=pl.BlockSpec((G, D), _fixed_spec),
    out_shape=jax.ShapeDtypeStruct((G, D), _f32),
    scratch_shapes=[pltpu.VMEM((G, D), _f32)],
)


def _bc8(v):
    return jnp.broadcast_to(v[None, :], (8, v.shape[0]))


@jax.jit
def kernel(x, edge_index, batch, W_first, b_first, W_hidden, b_hidden,
           W_last, b_last, bn_gamma, bn_beta, fc1_W, fc1_b, fc2_W, fc2_b):
    i32 = jnp.int32
    pad_e = jnp.full((EP - E,), N, i32)
    srcp = jnp.concatenate([edge_index[0], pad_e]).reshape(EP // CH, CH)
    dstp = jnp.concatenate([edge_index[1], pad_e]).reshape(EP // CH, CH)
    x_pad = jnp.pad(x, ((0, NP - N), (0, 0)))
    batch_bc = jnp.broadcast_to(
        jnp.pad(batch, (0, NP - N), constant_values=G)[:, None], (NP, D))
    zerosN = jnp.zeros((NP // 16, 16), _f32)
    srcp2 = srcp.reshape(EP // CH3, CH3)
    dstp2 = dstp.reshape(EP // CH3, CH3)

    degp = _sc_deg(dstp, zerosN)
    d4 = degp.reshape(NW, NBLK, BN // D, D)
    ysc, isqb = _tc_first(x_pad, d4, W_first)

    bs = [b_first] + [b_hidden[k] for k in range(4)]
    ws = [W_hidden[k] for k in range(4)] + [W_last]
    for l in range(5):
        p = _sc_scatter(ysc, srcp2, dstp2)
        ysc = _tc_mid(p[0], p[1], ysc, isqb, _bc8(bs[l]), ws[l])
    p = _sc_scatter(ysc, srcp2, dstp2)

    f2w_pad = jnp.zeros((D, D), _f32).at[:, :NCLS].set(fc2_W)
    f2b_pad = jnp.zeros((D,), _f32).at[:NCLS].set(fc2_b)
    res = _tc_final(p[0], p[1], ysc, isqb, _bc8(b_last), batch_bc,
                    _bc8(bn_gamma), _bc8(bn_beta), fc1_W, _bc8(fc1_b),
                    f2w_pad, _bc8(f2b_pad))
    return res[:, :NCLS]
